# baseline (speedup 1.0000x reference)
import functools

import jax
import jax.numpy as jnp
from jax.experimental import pallas as pl
from jax.experimental.pallas import tpu as pltpu

_GRID = (32, 32)
_ALPHA = 1.0
_N_NODES = _GRID[0] * _GRID[1]
_BLK = 512

_DN_T = (((1,), (1,)), ((), ()))


def _som_block(tw_base, z_ref, tw_ref, nodes_ref, som_ref, q_ref, idx_ref,
               nodes_t_ref, nn_ref, tw_col_ref):
    @pl.when(pl.program_id(0) == 0)
    def _prologue():
        nt = jnp.transpose(nodes_ref[...], (1, 0))
        nodes_t_ref[...] = nt
        nn_ref[...] = jnp.sum(nt * nt, axis=0, keepdims=True)
        tw_row = tw_ref[:, pl.ds(tw_base, _BLK)]
        tw_col_ref[...] = jnp.transpose(tw_row, (1, 0))

    nodes_t = nodes_t_ref[...]
    nn = nn_ref[...]
    h = _BLK // 2
    idx_parts = []
    for s in range(2):
        rows = pl.ds(s * h, h)
        z = z_ref[rows, :]
        tw = tw_col_ref[rows, :]
        wz = z * tw

        mm = jnp.dot(wz, nodes_t, preferred_element_type=jnp.float32)
        rowsq = jnp.sum(wz * wz, axis=1, keepdims=True)
        sq = rowsq - 2.0 * mm + nn
        dists = jnp.sqrt(jnp.maximum(sq, 1e-12))

        q_raw = 1.0 / (1.0 + dists / _ALPHA)
        q_ref[rows, :] = q_raw / jnp.sum(q_raw, axis=1, keepdims=True)

        idx = jnp.argmin(dists, axis=1).astype(jnp.int32)
        idx_parts.append(idx)

        lane = jax.lax.broadcasted_iota(jnp.int32, dists.shape, 1)
        onehot = (lane == idx[:, None]).astype(jnp.float32)
        gathered = jax.lax.dot_general(onehot, nodes_t, _DN_T,
                                       preferred_element_type=jnp.float32)
        som_ref[rows, :] = 0.9 * z + 0.1 * gathered

    idx_all = jnp.concatenate(idx_parts)
    idx_ref[...] = idx_all[None, None, :]


@jax.jit
def kernel(z, nodes, time_weights):
    b, t, d = z.shape
    n_rows = b * t
    z_flat = z.reshape(n_rows, d)
    nodes_flat = nodes.reshape(-1, d)
    max_seq = time_weights.shape[1]
    tw_row = time_weights.reshape(1, max_seq)
    assert t == _BLK, "row blocks must align with the sequence length"

    n_blocks = n_rows // _BLK

    som, q, idx = pl.pallas_call(
        functools.partial(_som_block, max_seq - t),
        grid=(n_blocks,),
        in_specs=[
            pl.BlockSpec((_BLK, d), lambda i: (i, 0)),
            pl.BlockSpec((1, max_seq), lambda i: (0, 0)),
            pl.BlockSpec((_N_NODES, d), lambda i: (0, 0)),
        ],
        out_specs=[
            pl.BlockSpec((_BLK, d), lambda i: (i, 0)),
            pl.BlockSpec((_BLK, _N_NODES), lambda i: (i, 0)),
            pl.BlockSpec((1, 1, _BLK), lambda i: (i, 0, 0)),
        ],
        out_shape=[
            jax.ShapeDtypeStruct((n_rows, d), jnp.float32),
            jax.ShapeDtypeStruct((n_rows, _N_NODES), jnp.float32),
            jax.ShapeDtypeStruct((n_blocks, 1, _BLK), jnp.int32),
        ],
        scratch_shapes=[
            pltpu.VMEM((d, _N_NODES), jnp.float32),
            pltpu.VMEM((1, _N_NODES), jnp.float32),
            pltpu.VMEM((_BLK, 1), jnp.float32),
        ],
    )(z_flat, tw_row, nodes_flat)

    som_z = som.reshape(b, t, d)
    bmu_indices = idx.reshape(b, t)
    return som_z, q, bmu_indices


# speedup vs baseline: 1.0621x; 1.0621x over previous
import functools

import jax
import jax.numpy as jnp
from jax.experimental import pallas as pl
from jax.experimental.pallas import tpu as pltpu

_GRID = (32, 32)
_ALPHA = 1.0
_N_NODES = _GRID[0] * _GRID[1]
_BLK = 512

_DN_T = (((1,), (1,)), ((), ()))


def _som_block(tw_base, z_ref, tw_ref, nodes_ref, som_ref, q_ref, idx_ref,
               nodes_t_m2_ref, nodes_t01_ref, nn_ref, tw_col_ref):
    @pl.when(pl.program_id(0) == 0)
    def _prologue():
        nt = jnp.transpose(nodes_ref[...], (1, 0))
        nodes_t_m2_ref[...] = -2.0 * nt
        nodes_t01_ref[...] = 0.1 * nt
        nn_ref[...] = jnp.sum(nt * nt, axis=0, keepdims=True)
        tw_row = tw_ref[:, pl.ds(tw_base, _BLK)]
        tw_col_ref[...] = jnp.transpose(tw_row, (1, 0))

    z = z_ref[...]
    tw = tw_col_ref[...]
    wz = z * tw

    mm2 = jnp.dot(wz, nodes_t_m2_ref[...],
                  preferred_element_type=jnp.float32)
    rowsq = jnp.sum(wz * wz, axis=1, keepdims=True)
    sq = rowsq + mm2 + nn_ref[...]
    dists = jnp.sqrt(jnp.maximum(sq, 1e-12))

    q_raw = 1.0 / (1.0 + dists / _ALPHA)
    q_ref[...] = q_raw / jnp.sum(q_raw, axis=1, keepdims=True)

    idx = jnp.argmin(dists, axis=1).astype(jnp.int32)
    idx_col = idx[:, None]
    idx_ref[...] = idx[None, None, :]

    lane = jax.lax.broadcasted_iota(jnp.int32, dists.shape, 1)
    onehot = (lane == idx_col).astype(jnp.float32)
    gathered01 = jax.lax.dot_general(onehot, nodes_t01_ref[...], _DN_T,
                                     preferred_element_type=jnp.float32)
    som_ref[...] = 0.9 * z + gathered01


@jax.jit
def kernel(z, nodes, time_weights):
    b, t, d = z.shape
    n_rows = b * t
    z_flat = z.reshape(n_rows, d)
    nodes_flat = nodes.reshape(-1, d)
    max_seq = time_weights.shape[1]
    tw_row = time_weights.reshape(1, max_seq)
    assert t == _BLK, "row blocks must align with the sequence length"

    n_blocks = n_rows // _BLK

    som, q, idx = pl.pallas_call(
        functools.partial(_som_block, max_seq - t),
        grid=(n_blocks,),
        in_specs=[
            pl.BlockSpec((_BLK, d), lambda i: (i, 0)),
            pl.BlockSpec((1, max_seq), lambda i: (0, 0)),
            pl.BlockSpec((_N_NODES, d), lambda i: (0, 0)),
        ],
        out_specs=[
            pl.BlockSpec((_BLK, d), lambda i: (i, 0)),
            pl.BlockSpec((_BLK, _N_NODES), lambda i: (i, 0)),
            pl.BlockSpec((1, 1, _BLK), lambda i: (i, 0, 0)),
        ],
        out_shape=[
            jax.ShapeDtypeStruct((n_rows, d), jnp.float32),
            jax.ShapeDtypeStruct((n_rows, _N_NODES), jnp.float32),
            jax.ShapeDtypeStruct((n_blocks, 1, _BLK), jnp.int32),
        ],
        scratch_shapes=[
            pltpu.VMEM((d, _N_NODES), jnp.float32),
            pltpu.VMEM((d, _N_NODES), jnp.float32),
            pltpu.VMEM((1, _N_NODES), jnp.float32),
            pltpu.VMEM((_BLK, 1), jnp.float32),
        ],
    )(z_flat, tw_row, nodes_flat)

    som_z = som.reshape(b, t, d)
    bmu_indices = idx.reshape(b, t)
    return som_z, q, bmu_indices


# speedup vs baseline: 1.0714x; 1.0088x over previous
import functools

import jax
import jax.numpy as jnp
from jax.experimental import pallas as pl
from jax.experimental.pallas import tpu as pltpu

_GRID = (32, 32)
_ALPHA = 1.0
_N_NODES = _GRID[0] * _GRID[1]
_BLK = 512

_DN_T = (((1,), (1,)), ((), ()))


def _som_block(tw_base, z_ref, tw_ref, nodes_ref, som_ref, q_ref, idx_ref,
               nodes_t_ref, nn_ref, tw_col_ref):
    @pl.when(pl.program_id(0) == 0)
    def _prologue():
        nt = jnp.transpose(nodes_ref[...], (1, 0))
        nodes_t_ref[...] = nt
        nn_ref[...] = jnp.sum(nt * nt, axis=0, keepdims=True)
        tw_row = tw_ref[:, pl.ds(tw_base, _BLK)]
        tw_col_ref[...] = jnp.transpose(tw_row, (1, 0))

    z = z_ref[...]
    tw = tw_col_ref[...]
    nodes_t = nodes_t_ref[...]
    wz = z * tw

    mm = jnp.dot(wz, nodes_t, preferred_element_type=jnp.float32)
    rowsq = jnp.sum(wz * wz, axis=1, keepdims=True)
    sq = rowsq - 2.0 * mm + nn_ref[...]
    dists = jnp.sqrt(jnp.maximum(sq, 1e-12))

    q_raw = 1.0 / (1.0 + (dists if _ALPHA == 1.0 else dists / _ALPHA))
    q_ref[...] = q_raw / jnp.sum(q_raw, axis=1, keepdims=True)

    idx = jnp.argmin(dists, axis=1).astype(jnp.int32)
    idx_col = idx[:, None]
    idx_ref[...] = idx[None, None, :]

    lane = jax.lax.broadcasted_iota(jnp.int32, dists.shape, 1)
    onehot = (lane == idx_col).astype(jnp.float32)
    gathered = jax.lax.dot_general(onehot, nodes_t, _DN_T,
                                   preferred_element_type=jnp.float32)
    som_ref[...] = 0.9 * z + 0.1 * gathered


@jax.jit
def kernel(z, nodes, time_weights):
    b, t, d = z.shape
    n_rows = b * t
    z_flat = z.reshape(n_rows, d)
    nodes_flat = nodes.reshape(-1, d)
    max_seq = time_weights.shape[1]
    tw_row = time_weights.reshape(1, max_seq)
    assert t == _BLK, "row blocks must align with the sequence length"

    n_blocks = n_rows // _BLK

    som, q, idx = pl.pallas_call(
        functools.partial(_som_block, max_seq - t),
        grid=(n_blocks,),
        in_specs=[
            pl.BlockSpec((_BLK, d), lambda i: (i, 0)),
            pl.BlockSpec((1, max_seq), lambda i: (0, 0)),
            pl.BlockSpec((_N_NODES, d), lambda i: (0, 0)),
        ],
        out_specs=[
            pl.BlockSpec((_BLK, d), lambda i: (i, 0)),
            pl.BlockSpec((_BLK, _N_NODES), lambda i: (i, 0)),
            pl.BlockSpec((1, 1, _BLK), lambda i: (i, 0, 0)),
        ],
        out_shape=[
            jax.ShapeDtypeStruct((n_rows, d), jnp.float32),
            jax.ShapeDtypeStruct((n_rows, _N_NODES), jnp.float32),
            jax.ShapeDtypeStruct((n_blocks, 1, _BLK), jnp.int32),
        ],
        scratch_shapes=[
            pltpu.VMEM((d, _N_NODES), jnp.float32),
            pltpu.VMEM((1, _N_NODES), jnp.float32),
            pltpu.VMEM((_BLK, 1), jnp.float32),
        ],
    )(z_flat, tw_row, nodes_flat)

    som_z = som.reshape(b, t, d)
    bmu_indices = idx.reshape(b, t)
    return som_z, q, bmu_indices
